# baseline (speedup 1.0000x reference)
import functools

import jax
import jax.numpy as jnp
from jax import lax
from jax.experimental import pallas as pl
from jax.experimental.pallas import tpu as pltpu
from jax.experimental.pallas import tpu_sc as plsc

B, N, D = 16, 2048, 2048
ROWS = B * N

NC, NS, L = 2, 16, 16
NW = NC * NS
RPW = ROWS // NW
CH = 16
NCH = RPW // CH
RG = 8
NJ = D // L

def _lane_gather(a, idx):
    return lax.gather(
        a, idx[:, None],
        lax.GatherDimensionNumbers(
            offset_dims=(), collapsed_slice_dims=(0,), start_index_map=(0,)),
        slice_sizes=(1,),
        mode=lax.GatherScatterMode.PROMISE_IN_BOUNDS)


_mesh = plsc.VectorSubcoreMesh(
    core_axis_name="c", subcore_axis_name="s", num_cores=NC, num_subcores=NS)


@functools.partial(
    pl.kernel,
    out_type=jax.ShapeDtypeStruct((ROWS,), jnp.float32),
    mesh=_mesh,
    scratch_types=[
        pltpu.VMEM((D,), jnp.float32),
        pltpu.VMEM((2, CH * D), jnp.float32),
        pltpu.VMEM((RPW,), jnp.float32),
        pltpu.SemaphoreType.DMA,
        pltpu.SemaphoreType.DMA,
    ],
)
def _sc_scores(x_hbm, w_hbm, out_hbm, w_v, bufs, sc_v, sem0, sem1):
    wid = lax.axis_index("s") * NC + lax.axis_index("c")
    base_row = wid * RPW
    sems = (sem0, sem1)
    lane = lax.broadcasted_iota(jnp.int32, (L,), 0)

    pltpu.sync_copy(w_hbm, w_v)

    def chunk_src(c):
        return x_hbm.at[pl.ds((base_row + c * CH) * D, CH * D)]

    for b in (0, 1):
        pltpu.async_copy(chunk_src(b), bufs.at[b], sems[b])

    def outer(g, _):
        for b in (0, 1):
            c = 2 * g + b
            pltpu.make_async_copy(chunk_src(c), bufs.at[b], sems[b]).wait()

            sv = jnp.zeros((L,), jnp.float32)
            for rg in range(CH // RG):
                def jbody(j, accs, _rg=rg, _b=b):
                    wv = w_v[pl.ds(j * L, L)]
                    return tuple(
                        accs[r] + bufs[_b, pl.ds((_rg * RG + r) * D + j * L, L)] * wv
                        for r in range(RG))
                accs = lax.fori_loop(
                    0, NJ, jbody,
                    tuple(jnp.zeros((L,), jnp.float32) for _ in range(RG)))
                for r in range(RG):
                    a = accs[r]
                    for sft in (8, 4, 2, 1):
                        a = a + _lane_gather(a, lane ^ sft)
                    sv = jnp.where(lane == (rg * RG + r), a, sv)
            sc_v[pl.ds(c * CH, CH)] = sv

            @pl.when(c + 2 < NCH)
            def _():
                pltpu.async_copy(chunk_src(c + 2), bufs.at[b], sems[b])
        return _

    lax.fori_loop(0, NCH // 2, outer, None)
    pltpu.sync_copy(sc_v, out_hbm.at[pl.ds(base_row, RPW)])


def _head_body(s_ref, w1t_ref, b1_ref, w2t_ref, b2_ref, w3t_ref, b3_ref, o_ref):
    s = s_ref[...]
    iota = jax.lax.broadcasted_iota(jnp.int32, (B, N), 1)

    max1 = jnp.max(s, axis=1, keepdims=True)
    idx_max = jnp.min(jnp.where(s == max1, iota, N), axis=1, keepdims=True)
    max2 = jnp.max(jnp.where(iota == idx_max, -jnp.inf, s), axis=1, keepdims=True)

    min1 = jnp.min(s, axis=1, keepdims=True)
    idx_min = jnp.min(jnp.where(s == min1, iota, N), axis=1, keepdims=True)
    min2 = jnp.min(jnp.where(iota == idx_min, jnp.inf, s), axis=1, keepdims=True)

    f = jnp.concatenate([min1, min2, max1, max2], axis=1)

    h = b1_ref[...]
    w1t = w1t_ref[...]
    for k in range(4):
        h = h + f[:, k:k + 1] * w1t[k:k + 1, :]
    h = jax.nn.sigmoid(h)

    h2 = jax.nn.sigmoid(
        jax.lax.dot_general(h, w2t_ref[...],
                            dimension_numbers=(((1,), (0,)), ((), ())),
                            preferred_element_type=jnp.float32)
        + b2_ref[...])

    o_ref[...] = jax.nn.sigmoid(
        jax.lax.dot_general(h2, w3t_ref[...],
                            dimension_numbers=(((1,), (0,)), ((), ())),
                            preferred_element_type=jnp.float32)
        + b3_ref[...])


@jax.jit
def kernel(x, W_embed, W1, b1, W2, b2, W3, b3):
    scores = _sc_scores(x.reshape(ROWS * D), W_embed.reshape(D))

    out = pl.pallas_call(
        _head_body,
        out_shape=jax.ShapeDtypeStruct((B, 1), jnp.float32),
    )(
        scores.reshape(B, N),
        W1.T, b1.reshape(1, 200),
        W2.T, b2.reshape(1, 100),
        W3.T, b3.reshape(1, 1),
    )
    return out.reshape(-1)


# speedup vs baseline: 2.7687x; 2.7687x over previous
import functools

import jax
import jax.numpy as jnp
from jax import lax
from jax.experimental import pallas as pl
from jax.experimental.pallas import tpu as pltpu
from jax.experimental.pallas import tpu_sc as plsc

B, N, D = 16, 2048, 2048
ROWS = B * N

NC, NS, L = 2, 16, 16
NW = NC * NS
RPW = ROWS // NW
CH = 16
NCH = RPW // CH
RG = 8
NJ = D // L

def _lane_gather(a, idx):
    return lax.gather(
        a, idx[:, None],
        lax.GatherDimensionNumbers(
            offset_dims=(), collapsed_slice_dims=(0,), start_index_map=(0,)),
        slice_sizes=(1,),
        mode=lax.GatherScatterMode.PROMISE_IN_BOUNDS)


_mesh = plsc.VectorSubcoreMesh(
    core_axis_name="c", subcore_axis_name="s", num_cores=NC, num_subcores=NS)


@functools.partial(
    pl.kernel,
    out_type=jax.ShapeDtypeStruct((ROWS,), jnp.float32),
    mesh=_mesh,
    scratch_types=[
        pltpu.VMEM((D,), jnp.float32),
        pltpu.VMEM((2, CH, D), jnp.float32),
        pltpu.VMEM((RPW,), jnp.float32),
        pltpu.SemaphoreType.DMA,
        pltpu.SemaphoreType.DMA,
    ],
)
def _sc_scores(x_hbm, w_hbm, out_hbm, w_v, bufs, sc_v, sem0, sem1):
    wid = lax.axis_index("s") * NC + lax.axis_index("c")
    base_row = wid * RPW
    sems = (sem0, sem1)
    lane = lax.broadcasted_iota(jnp.int32, (L,), 0)

    pltpu.sync_copy(w_hbm, w_v)

    def chunk_src(c):
        return x_hbm.at[pl.ds(base_row + c * CH, CH), :]

    for b in (0, 1):
        pltpu.async_copy(chunk_src(b), bufs.at[b], sems[b])

    def outer(g, _):
        for b in (0, 1):
            c = 2 * g + b
            pltpu.make_async_copy(chunk_src(c), bufs.at[b], sems[b]).wait()

            sv = jnp.zeros((L,), jnp.float32)
            for rg in range(CH // RG):
                def jbody(j, accs, _rg=rg, _b=b):
                    wv = w_v[pl.ds(j * L, L)]
                    return tuple(
                        accs[r] + bufs[_b, _rg * RG + r, pl.ds(j * L, L)] * wv
                        for r in range(RG))
                accs = lax.fori_loop(
                    0, NJ, jbody,
                    tuple(jnp.zeros((L,), jnp.float32) for _ in range(RG)),
                    unroll=8)
                for r in range(RG):
                    a = accs[r]
                    for sft in (8, 4, 2, 1):
                        a = a + _lane_gather(a, lane ^ sft)
                    sv = jnp.where(lane == (rg * RG + r), a, sv)
            sc_v[pl.ds(c * CH, CH)] = sv

            @pl.when(c + 2 < NCH)
            def _():
                pltpu.async_copy(chunk_src(c + 2), bufs.at[b], sems[b])
        return _

    lax.fori_loop(0, NCH // 2, outer, None)
    pltpu.sync_copy(sc_v, out_hbm.at[pl.ds(base_row, RPW)])


def _head_body(s_ref, w1t_ref, b1_ref, w2t_ref, b2_ref, w3t_ref, b3_ref, o_ref):
    s = s_ref[...]
    iota = jax.lax.broadcasted_iota(jnp.int32, (B, N), 1)

    max1 = jnp.max(s, axis=1, keepdims=True)
    idx_max = jnp.min(jnp.where(s == max1, iota, N), axis=1, keepdims=True)
    max2 = jnp.max(jnp.where(iota == idx_max, -jnp.inf, s), axis=1, keepdims=True)

    min1 = jnp.min(s, axis=1, keepdims=True)
    idx_min = jnp.min(jnp.where(s == min1, iota, N), axis=1, keepdims=True)
    min2 = jnp.min(jnp.where(iota == idx_min, jnp.inf, s), axis=1, keepdims=True)

    f = jnp.concatenate([min1, min2, max1, max2], axis=1)

    h = b1_ref[...]
    w1t = w1t_ref[...]
    for k in range(4):
        h = h + f[:, k:k + 1] * w1t[k:k + 1, :]
    h = jax.nn.sigmoid(h)

    h2 = jax.nn.sigmoid(
        jax.lax.dot_general(h, w2t_ref[...],
                            dimension_numbers=(((1,), (0,)), ((), ())),
                            preferred_element_type=jnp.float32)
        + b2_ref[...])

    o_ref[...] = jax.nn.sigmoid(
        jax.lax.dot_general(h2, w3t_ref[...],
                            dimension_numbers=(((1,), (0,)), ((), ())),
                            preferred_element_type=jnp.float32)
        + b3_ref[...])


@jax.jit
def kernel(x, W_embed, W1, b1, W2, b2, W3, b3):
    scores = _sc_scores(x.reshape(ROWS, D), W_embed.reshape(D))

    out = pl.pallas_call(
        _head_body,
        out_shape=jax.ShapeDtypeStruct((B, 1), jnp.float32),
    )(
        scores.reshape(B, N),
        W1.T, b1.reshape(1, 200),
        W2.T, b2.reshape(1, 100),
        W3.T, b3.reshape(1, 1),
    )
    return out.reshape(-1)


# speedup vs baseline: 3.4455x; 1.2445x over previous
import functools

import jax
import jax.numpy as jnp
from jax import lax
from jax.experimental import pallas as pl
from jax.experimental.pallas import tpu as pltpu
from jax.experimental.pallas import tpu_sc as plsc

B, N, D = 16, 2048, 2048
ROWS = B * N
SB = 8

NC, NS, L = 2, 16, 16
NW = NC * NS
SC_ROWS = SB * N
RPW = SC_ROWS // NW
CH = 16
NCH = RPW // CH
RG = 8
NJ = D // L


def _lane_gather(a, idx):
    return lax.gather(
        a, idx[:, None],
        lax.GatherDimensionNumbers(
            offset_dims=(), collapsed_slice_dims=(0,), start_index_map=(0,)),
        slice_sizes=(1,),
        mode=lax.GatherScatterMode.PROMISE_IN_BOUNDS)


_mesh = plsc.VectorSubcoreMesh(
    core_axis_name="c", subcore_axis_name="s", num_cores=NC, num_subcores=NS)


@functools.partial(
    pl.kernel,
    out_type=jax.ShapeDtypeStruct((SC_ROWS,), jnp.float32),
    mesh=_mesh,
    scratch_types=[
        pltpu.VMEM((D,), jnp.float32),
        pltpu.VMEM((2, CH, D), jnp.float32),
        pltpu.VMEM((RPW,), jnp.float32),
        pltpu.SemaphoreType.DMA,
        pltpu.SemaphoreType.DMA,
    ],
)
def _sc_scores(x_hbm, w_hbm, out_hbm, w_v, bufs, sc_v, sem0, sem1):
    wid = lax.axis_index("s") * NC + lax.axis_index("c")
    base_row = wid * RPW
    sems = (sem0, sem1)
    lane = lax.broadcasted_iota(jnp.int32, (L,), 0)

    pltpu.sync_copy(w_hbm, w_v)

    def chunk_src(c):
        return x_hbm.at[pl.ds(base_row + c * CH, CH), :]

    for b in (0, 1):
        pltpu.async_copy(chunk_src(b), bufs.at[b], sems[b])

    def outer(g, _):
        for b in (0, 1):
            c = 2 * g + b
            pltpu.make_async_copy(chunk_src(c), bufs.at[b], sems[b]).wait()

            sv = jnp.zeros((L,), jnp.float32)
            for rg in range(CH // RG):
                def jbody(j, accs, _rg=rg, _b=b):
                    wv = w_v[pl.ds(j * L, L)]
                    return tuple(
                        accs[r] + bufs[_b, _rg * RG + r, pl.ds(j * L, L)] * wv
                        for r in range(RG))
                accs = lax.fori_loop(
                    0, NJ, jbody,
                    tuple(jnp.zeros((L,), jnp.float32) for _ in range(RG)),
                    unroll=8)
                for r in range(RG):
                    a = accs[r]
                    for sft in (8, 4, 2, 1):
                        a = a + _lane_gather(a, lane ^ sft)
                    sv = jnp.where(lane == (rg * RG + r), a, sv)
            sc_v[pl.ds(c * CH, CH)] = sv

            @pl.when(c + 2 < NCH)
            def _():
                pltpu.async_copy(chunk_src(c + 2), bufs.at[b], sems[b])
        return _

    lax.fori_loop(0, NCH // 2, outer, None)
    pltpu.sync_copy(sc_v, out_hbm.at[pl.ds(base_row, RPW)])


def _select_mlp(s, iota, w1t_ref, b1_ref, w2t_ref, b2_ref, w3t_ref, b3_ref):
    max1 = jnp.max(s)
    idx_max = jnp.min(jnp.where(s == max1, iota, iota.size))
    max2 = jnp.max(jnp.where(iota == idx_max, -jnp.inf, s))

    min1 = jnp.min(s)
    idx_min = jnp.min(jnp.where(s == min1, iota, iota.size))
    min2 = jnp.min(jnp.where(iota == idx_min, jnp.inf, s))

    h = (b1_ref[...]
         + min1 * w1t_ref[0:1, :]
         + min2 * w1t_ref[1:2, :]
         + max1 * w1t_ref[2:3, :]
         + max2 * w1t_ref[3:4, :])
    h = jax.nn.sigmoid(h)

    h2 = jax.nn.sigmoid(
        jax.lax.dot_general(h, w2t_ref[...],
                            dimension_numbers=(((1,), (0,)), ((), ())),
                            preferred_element_type=jnp.float32)
        + b2_ref[...])

    return jax.nn.sigmoid(
        jax.lax.dot_general(h2, w3t_ref[...],
                            dimension_numbers=(((1,), (0,)), ((), ())),
                            preferred_element_type=jnp.float32)
        + b3_ref[...])


def _tc_body(x_ref, w_ref, w1t_ref, b1_ref, w2t_ref, b2_ref, w3t_ref, b3_ref,
             o_ref):
    s = jax.lax.dot_general(
        x_ref[...], w_ref[...],
        dimension_numbers=(((1,), (0,)), ((), ())),
        preferred_element_type=jnp.float32,
    )
    iota = jax.lax.broadcasted_iota(jnp.int32, (N, 1), 0)
    i = pl.program_id(0)
    o_ref[pl.ds(i, 1), :] = _select_mlp(
        s, iota, w1t_ref, b1_ref, w2t_ref, b2_ref, w3t_ref, b3_ref)


def _head_body(s_ref, w1t_ref, b1_ref, w2t_ref, b2_ref, w3t_ref, b3_ref,
               o_ref):
    s = s_ref[...]
    iota = jax.lax.broadcasted_iota(jnp.int32, (SB, N), 1)

    max1 = jnp.max(s, axis=1, keepdims=True)
    idx_max = jnp.min(jnp.where(s == max1, iota, N), axis=1, keepdims=True)
    max2 = jnp.max(jnp.where(iota == idx_max, -jnp.inf, s), axis=1,
                   keepdims=True)

    min1 = jnp.min(s, axis=1, keepdims=True)
    idx_min = jnp.min(jnp.where(s == min1, iota, N), axis=1, keepdims=True)
    min2 = jnp.min(jnp.where(iota == idx_min, jnp.inf, s), axis=1,
                   keepdims=True)

    f = jnp.concatenate([min1, min2, max1, max2], axis=1)

    h = b1_ref[...]
    w1t = w1t_ref[...]
    for k in range(4):
        h = h + f[:, k:k + 1] * w1t[k:k + 1, :]
    h = jax.nn.sigmoid(h)

    h2 = jax.nn.sigmoid(
        jax.lax.dot_general(h, w2t_ref[...],
                            dimension_numbers=(((1,), (0,)), ((), ())),
                            preferred_element_type=jnp.float32)
        + b2_ref[...])

    o_ref[...] = jax.nn.sigmoid(
        jax.lax.dot_general(h2, w3t_ref[...],
                            dimension_numbers=(((1,), (0,)), ((), ())),
                            preferred_element_type=jnp.float32)
        + b3_ref[...])


@jax.jit
def kernel(x, W_embed, W1, b1, W2, b2, W3, b3):
    xf = x.reshape(ROWS, D)
    wt = W_embed.reshape(D, 1)
    w1t = W1.T
    b1r = b1.reshape(1, 200)
    w2t = W2.T
    b2r = b2.reshape(1, 100)
    w3t = W3.T
    b3r = b3.reshape(1, 1)
    const = lambda i: (0, 0)

    sc_scores = _sc_scores(xf, W_embed.reshape(D))

    out_tc = pl.pallas_call(
        _tc_body,
        grid=(B - SB,),
        in_specs=[
            pl.BlockSpec((N, D), lambda i: (SB + i, 0)),
            pl.BlockSpec((D, 1), const),
            pl.BlockSpec((4, 200), const),
            pl.BlockSpec((1, 200), const),
            pl.BlockSpec((200, 100), const),
            pl.BlockSpec((1, 100), const),
            pl.BlockSpec((100, 1), const),
            pl.BlockSpec((1, 1), const),
        ],
        out_specs=pl.BlockSpec((B - SB, 1), const),
        out_shape=jax.ShapeDtypeStruct((B - SB, 1), jnp.float32),
    )(xf, wt, w1t, b1r, w2t, b2r, w3t, b3r)

    out_sc = pl.pallas_call(
        _head_body,
        out_shape=jax.ShapeDtypeStruct((SB, 1), jnp.float32),
    )(sc_scores.reshape(SB, N), w1t, b1r, w2t, b2r, w3t, b3r)

    return jnp.concatenate([out_sc, out_tc], axis=0).reshape(-1)
